# baseline (speedup 1.0000x reference)
import functools
from functools import partial

import jax
import jax.numpy as jnp
from jax.experimental import pallas as pl
from jax.experimental.pallas import tpu as pltpu

N_NODES = 10000
N_EDGES = 160000
D = 256
D_EDGE = 768
BATCH_SIZE = 128.0
EPS = 1e-08


def _mm_nt_body(a_ref, b_ref, o_ref):
    o_ref[...] = jax.lax.dot_general(
        a_ref[...], b_ref[...],
        dimension_numbers=(((1,), (1,)), ((), ())),
        preferred_element_type=jnp.float32)


def _matmul_nt(a, b, bm):
    m, k = a.shape
    n = b.shape[0]
    grid = (m // bm,)
    return pl.pallas_call(
        _mm_nt_body,
        grid=grid,
        in_specs=[pl.BlockSpec((bm, k), lambda i: (i, 0)),
                  pl.BlockSpec((n, k), lambda i: (0, 0))],
        out_specs=pl.BlockSpec((bm, n), lambda i: (i, 0)),
        out_shape=jax.ShapeDtypeStruct((m, n), jnp.float32),
    )(a, b)


def _msg_body(gi_ref, gj_ref, e_ref, w1_ref, t_ref, ps_ref):
    v = gi_ref[...] + gj_ref[...] + e_ref[...]
    u = jnp.maximum(v, 0.0)
    s = jnp.sum(u * w1_ref[...], axis=1)
    t = jnp.exp(s)
    t_ref[...] = t[None, :]

    @pl.when(pl.program_id(0) == 0)
    def _init():
        ps_ref[...] = jnp.zeros((1, 1), jnp.float32)

    ps_ref[...] += jnp.sum(t).reshape(1, 1)


def _msg_scalar(gi, gj, e, w1, bm):
    m = gi.shape[0]
    grid = (m // bm,)
    t, ps = pl.pallas_call(
        _msg_body,
        grid=grid,
        in_specs=[pl.BlockSpec((bm, D), lambda i: (i, 0)),
                  pl.BlockSpec((bm, D), lambda i: (i, 0)),
                  pl.BlockSpec((bm, D), lambda i: (i, 0)),
                  pl.BlockSpec((1, D), lambda i: (0, 0))],
        out_specs=[pl.BlockSpec((1, bm), lambda i: (0, i)),
                   pl.BlockSpec((1, 1), lambda i: (0, 0))],
        out_shape=[jax.ShapeDtypeStruct((1, m), jnp.float32),
                   jax.ShapeDtypeStruct((1, 1), jnp.float32)],
    )(gi, gj, e, w1)
    return t[0], ps[0]


def kernel(x, edge_index, edge_attr, params):
    src = edge_index[0]
    dst = edge_index[1]
    n = x.shape[0]

    layer_w = []
    for p in params['layers']:
        dirs = []
        for (lin, line, w0, w1) in (
            (p['lin1'], p['lin3'], p['lin5_0'], p['lin5_1']),
            (p['lin2'], p['lin4'], p['lin6_0'], p['lin6_1']),
        ):
            ci = w0[:, :D] @ lin
            cj = w0[:, D:2 * D] @ lin
            ce = w0[:, 2 * D:] @ line
            dirs.append({'lin': lin, 'ci': ci, 'cj': cj, 'ce': ce, 'w1': w1})
        layer_w.append(dirs)

    ce_all = jnp.concatenate(
        [layer_w[l][d]['ce'] for l in range(2) for d in range(2)], axis=0)
    e_all = _matmul_nt(edge_attr, ce_all, bm=640)

    counts = [None, None]
    ones = jnp.ones((N_EDGES, 1), jnp.float32)
    counts[0] = jax.ops.segment_sum(ones, dst, num_segments=n)
    counts[1] = jax.ops.segment_sum(ones, src, num_segments=n)

    xs = [x]
    for l, p in enumerate(params['layers']):
        xin = xs[-1]
        dirs = layer_w[l]
        wn = jnp.concatenate([
            dirs[0]['lin'], dirs[0]['ci'], dirs[0]['cj'],
            dirs[1]['lin'], dirs[1]['ci'], dirs[1]['cj'],
            p['root_W'],
        ], axis=0)
        proj = _matmul_nt(xin, wn, bm=400)
        h1 = proj[:, 0:D]
        p1i = proj[:, D:2 * D]
        p1j = proj[:, 2 * D:3 * D]
        h2 = proj[:, 3 * D:4 * D]
        p2i = proj[:, 4 * D:5 * D]
        p2j = proj[:, 5 * D:6 * D]
        root = proj[:, 6 * D:7 * D]

        out_sum = root + p['root_b']
        for d_idx, (h, pi, pj, idx_i, idx_j, cnt) in enumerate((
                (h1, p1i, p1j, dst, src, counts[0]),
                (h2, p2i, p2j, src, dst, counts[1]))):
            e = jax.lax.dynamic_slice_in_dim(e_all, (2 * l + d_idx) * D, D, 1)
            gi = pi[idx_i]
            gj = pj[idx_j]
            t, ps = _msg_scalar(gi, gj, e, dirs[d_idx]['w1'], bm=640)
            add = jnp.sum(ps) / BATCH_SIZE
            add = jnp.where(add == 0.0, EPS, add)
            t = jnp.minimum(t / add, 1.0)
            m = (t * p['W'])[:, None] * h[idx_j]
            seg = jax.ops.segment_sum(m, idx_i, num_segments=n)
            out_sum = out_sum + seg / jnp.maximum(cnt, 1.0)
        xs.append(jnp.maximum(out_sum, 0.0))

    cat = jnp.concatenate(xs, axis=-1)
    out = _matmul_nt(cat, params['final_W'], bm=400) + params['final_b']
    return out


# speedup vs baseline: 1.6578x; 1.6578x over previous
import functools
from functools import partial

import jax
import jax.numpy as jnp
from jax import lax
from jax.experimental import pallas as pl
from jax.experimental.pallas import tpu as pltpu
from jax.experimental.pallas import tpu_sc as plsc

N_NODES = 10000
N_EDGES = 160000
D = 256
D_EDGE = 768
BATCH_SIZE = 128.0
EPS = 1e-08

NC = 2
NS = 16
NW = NC * NS

CH = 128
HF = 64
NCHUNK = N_EDGES // CH
NPAD = 10240
ACC_W = 128


def _sc_mesh():
    return plsc.VectorSubcoreMesh(
        core_axis_name="c", subcore_axis_name="s",
        num_cores=NC, num_subcores=NS)


def _mm_nt_body(a_ref, b_ref, o_ref):
    o_ref[...] = jax.lax.dot_general(
        a_ref[...], b_ref[...],
        dimension_numbers=(((1,), (1,)), ((), ())),
        preferred_element_type=jnp.float32)


def _matmul_nt(a, b, bm):
    m, k = a.shape
    n = b.shape[0]
    grid = (m // bm,)
    return pl.pallas_call(
        _mm_nt_body,
        grid=grid,
        in_specs=[pl.BlockSpec((bm, k), lambda i: (i, 0)),
                  pl.BlockSpec((n, k), lambda i: (0, 0))],
        out_specs=pl.BlockSpec((bm, n), lambda i: (i, 0)),
        out_shape=jax.ShapeDtypeStruct((m, n), jnp.float32),
    )(a, b)


P1_ITER = (NCHUNK + NW - 1) // NW


def _phase1_body(pi_h, pj_h, e_h, ii_h, ij_h, w1_h,
                 t_h,
                 ii_c, ij_c, gi0, gj0, ev0, gi1, gj1, ev1,
                 w1_v, tr_v, t_c, sem0, sem1, sem_t):
    cid = lax.axis_index("c")
    sid = lax.axis_index("s")
    wid = cid * NS + sid

    pltpu.sync_copy(w1_h, w1_v)
    lane = lax.broadcasted_iota(jnp.int32, (16,), 0)

    def issue(cb, half, gi_v, gj_v, ev_v, sem):
        pltpu.async_copy(pi_h.at[ii_c.at[pl.ds(half * HF, HF)]], gi_v, sem)
        pltpu.async_copy(pj_h.at[ij_c.at[pl.ds(half * HF, HF)]], gj_v, sem)
        pltpu.async_copy(e_h.at[pl.ds(cb + half * HF, HF)], ev_v, sem)

    def wait(gi_v, gj_v, ev_v, sem):
        for dst in (gi_v, gj_v, ev_v):
            pltpu.make_async_copy(e_h.at[pl.ds(0, HF)], dst, sem).wait()

    def compute(half, gi_v, gj_v, ev_v):
        @pl.loop(0, HF // 16)
        def _grp(grp):
            @pl.loop(0, 16)
            def _edge(j):
                e = grp * 16 + j
                acc = jnp.zeros((16,), jnp.float32)
                for k in range(D // 16):
                    v = (gi_v[e, pl.ds(k * 16, 16)]
                         + gj_v[e, pl.ds(k * 16, 16)]
                         + ev_v[e, pl.ds(k * 16, 16)])
                    acc = acc + jnp.maximum(v, 0.0) * w1_v[pl.ds(k * 16, 16)]
                plsc.store_scatter(tr_v, [lane * 16 + j], acc)
            s = tr_v[pl.ds(0, 16)]
            for k in range(1, 16):
                s = s + tr_v[pl.ds(k * 16, 16)]
            plsc.store_scatter(
                t_c, [jnp.full((16,), half * HF + grp * 16, jnp.int32)
                      + lane], jnp.exp(s))

    @pl.loop(0, P1_ITER)
    def _chunk(i):
        ch = wid + i * NW

        @pl.when(ch < NCHUNK)
        def _():
            cb = ch * CH
            pltpu.sync_copy(ii_h.at[pl.ds(cb, CH)], ii_c)
            pltpu.sync_copy(ij_h.at[pl.ds(cb, CH)], ij_c)
            issue(cb, 0, gi0, gj0, ev0, sem0)
            issue(cb, 1, gi1, gj1, ev1, sem1)

            @pl.when(i > 0)
            def _():
                pltpu.make_async_copy(ii_h.at[pl.ds(0, CH)], t_c,
                                      sem_t).wait()

            wait(gi0, gj0, ev0, sem0)
            compute(0, gi0, gj0, ev0)
            wait(gi1, gj1, ev1, sem1)
            compute(1, gi1, gj1, ev1)
            pltpu.async_copy(t_c, t_h.at[pl.ds(cb, CH)], sem_t)

    pltpu.make_async_copy(ii_h.at[pl.ds(0, CH)], t_c, sem_t).wait()


def _phase1_call(pi, pj, e, idx_i, idx_j, w1vec):
    kern = pl.kernel(
        _phase1_body,
        out_type=jax.ShapeDtypeStruct((N_EDGES,), jnp.float32),
        mesh=_sc_mesh(),
        compiler_params=pltpu.CompilerParams(needs_layout_passes=False),
        scratch_types=[
            pltpu.VMEM((CH,), jnp.int32),
            pltpu.VMEM((CH,), jnp.int32),
            pltpu.VMEM((HF, D), jnp.float32),
            pltpu.VMEM((HF, D), jnp.float32),
            pltpu.VMEM((HF, D), jnp.float32),
            pltpu.VMEM((HF, D), jnp.float32),
            pltpu.VMEM((HF, D), jnp.float32),
            pltpu.VMEM((HF, D), jnp.float32),
            pltpu.VMEM((D,), jnp.float32),
            pltpu.VMEM((256,), jnp.float32),
            pltpu.VMEM((CH,), jnp.float32),
            pltpu.SemaphoreType.DMA,
            pltpu.SemaphoreType.DMA,
            pltpu.SemaphoreType.DMA,
        ],
    )
    return kern(pi, pj, e, idx_i, idx_j, w1vec)


P2_ITER = (NCHUNK + NS - 1) // NS


def _phase2_body(hcat_h, coef_h, ii_h, ij_h,
                 acc_h,
                 ii0, ii1, ij_c, cf_c, hv0, hv1, rb0, rb1,
                 acc_sp, semh0, semh1, sems0, sems1):
    cid = lax.axis_index("c")
    sid = lax.axis_index("s")

    rows = NPAD // NS
    zvec = jnp.zeros((16,), jnp.float32)

    @pl.loop(0, HF)
    def _zrow(e):
        for k in range(ACC_W // 16):
            rb0[e, pl.ds(k * 16, 16)] = zvec

    for r in range(rows // HF):
        pltpu.sync_copy(rb0, acc_sp.at[pl.ds(sid * rows + r * HF, HF)])

    plsc.subcore_barrier()

    off = jnp.full((16,), cid * N_NODES, jnp.int32)

    def compute(hv, rb, cf_base):
        @pl.loop(0, HF)
        def _edge(e):
            cvec = plsc.load_gather(
                cf_c, [jnp.full((16,), cf_base + e, jnp.int32)])
            for k in range(ACC_W // 16):
                rb[e, pl.ds(k * 16, 16)] = hv[e, pl.ds(k * 16, 16)] * cvec

    def drain_sc(rb, sems):
        pltpu.make_async_copy(hcat_h.at[pl.ds(0, HF)], rb, sems).wait()

    @pl.loop(0, P2_ITER)
    def _chunk(i):
        ch = sid + i * NS

        @pl.when(ch < NCHUNK)
        def _():
            cb = ch * CH
            pltpu.sync_copy(ij_h.at[pl.ds(cb, CH)], ij_c)
            pltpu.sync_copy(coef_h.at[pl.ds(cb, CH)], cf_c)
            pltpu.async_copy(ii_h.at[pl.ds(cb, HF)], ii0, semh0)
            pltpu.async_copy(ii_h.at[pl.ds(cb + HF, HF)], ii1, semh1)
            for r in range(CH // 16):
                ij_c[pl.ds(r * 16, 16)] = ij_c[pl.ds(r * 16, 16)] + off
            pltpu.async_copy(hcat_h.at[ij_c.at[pl.ds(0, HF)]], hv0, semh0)
            pltpu.async_copy(hcat_h.at[ij_c.at[pl.ds(HF, HF)]], hv1, semh1)

            pltpu.make_async_copy(ii_h.at[pl.ds(0, HF)], ii0, semh0).wait()
            pltpu.make_async_copy(hcat_h.at[pl.ds(0, HF)], hv0,
                                  semh0).wait()

            compute(hv0, rb0, 0)
            pltpu.sync_copy(rb0, acc_sp.at[ii0], add=True)

            pltpu.make_async_copy(ii_h.at[pl.ds(0, HF)], ii1, semh1).wait()
            pltpu.make_async_copy(hcat_h.at[pl.ds(0, HF)], hv1,
                                  semh1).wait()

            compute(hv1, rb1, HF)
            pltpu.sync_copy(rb1, acc_sp.at[ii1], add=True)

    plsc.subcore_barrier()

    pltpu.sync_copy(acc_sp.at[pl.ds(sid * rows, rows)],
                    acc_h.at[pl.ds(cid * NPAD + sid * rows, rows)])


def _phase2_call(hcat, coef, idx_i, idx_j):
    kern = pl.kernel(
        _phase2_body,
        out_type=jax.ShapeDtypeStruct((NC * NPAD, ACC_W), jnp.float32),
        mesh=_sc_mesh(),
        compiler_params=pltpu.CompilerParams(needs_layout_passes=False),
        scratch_types=[
            pltpu.VMEM((HF,), jnp.int32),
            pltpu.VMEM((HF,), jnp.int32),
            pltpu.VMEM((CH,), jnp.int32),
            pltpu.VMEM((CH,), jnp.float32),
            pltpu.VMEM((HF, 128), jnp.float32),
            pltpu.VMEM((HF, 128), jnp.float32),
            pltpu.VMEM((HF, ACC_W), jnp.float32),
            pltpu.VMEM((HF, ACC_W), jnp.float32),
            pltpu.VMEM_SHARED((NPAD, ACC_W), jnp.float32),
            pltpu.SemaphoreType.DMA,
            pltpu.SemaphoreType.DMA,
            pltpu.SemaphoreType.DMA,
            pltpu.SemaphoreType.DMA,
        ],
    )
    return kern(hcat, coef, idx_i, idx_j)


def kernel(x, edge_index, edge_attr, params):
    src = edge_index[0]
    dst = edge_index[1]
    n = x.shape[0]

    layer_w = []
    for p in params['layers']:
        dirs = []
        for (lin, line, w0, w1) in (
            (p['lin1'], p['lin3'], p['lin5_0'], p['lin5_1']),
            (p['lin2'], p['lin4'], p['lin6_0'], p['lin6_1']),
        ):
            ci = w0[:, :D] @ lin
            cj = w0[:, D:2 * D] @ lin
            ce = w0[:, 2 * D:] @ line
            dirs.append({'lin': lin, 'ci': ci, 'cj': cj, 'ce': ce,
                         'w1': w1[0]})
        layer_w.append(dirs)

    e_l = [
        _matmul_nt(edge_attr,
                   jnp.concatenate([layer_w[l][0]['ce'],
                                    layer_w[l][1]['ce']], axis=0),
                   bm=640)
        for l in range(len(params['layers']))
    ]

    ones = jnp.ones((N_EDGES, 1), jnp.float32)
    counts = [jnp.maximum(jax.ops.segment_sum(ones, dst, num_segments=n), 1.0),
              jnp.maximum(jax.ops.segment_sum(ones, src, num_segments=n), 1.0)]

    xs = [x]
    for l, p in enumerate(params['layers']):
        xin = xs[-1]
        dirs = layer_w[l]
        wn = jnp.concatenate([
            dirs[0]['lin'], dirs[0]['ci'], dirs[0]['cj'],
            dirs[1]['lin'], dirs[1]['ci'], dirs[1]['cj'],
            p['root_W'],
        ], axis=0)
        proj = _matmul_nt(xin, wn, bm=400)
        h1 = proj[:, 0:D]
        p1i = proj[:, D:2 * D]
        p1j = proj[:, 2 * D:3 * D]
        h2 = proj[:, 3 * D:4 * D]
        p2i = proj[:, 4 * D:5 * D]
        p2j = proj[:, 5 * D:6 * D]
        root = proj[:, 6 * D:7 * D]

        out_sum = root + p['root_b']
        for d_idx, (h, pi, pj, idx_i, idx_j) in enumerate((
                (h1, p1i, p1j, dst, src),
                (h2, p2i, p2j, src, dst))):
            e = jax.lax.dynamic_slice_in_dim(e_l[l], d_idx * D, D, 1)
            t = _phase1_call(pi, pj, e, idx_i, idx_j, dirs[d_idx]['w1'])
            add = jnp.sum(t) / BATCH_SIZE
            add = jnp.where(add == 0.0, EPS, add)
            coef = jnp.minimum(t / add, 1.0) * p['W']
            hcat = jnp.concatenate([h[:, :128], h[:, 128:]], axis=0)
            acc = _phase2_call(hcat, coef, idx_i, idx_j)
            seg = jnp.concatenate(
                [acc[:N_NODES, :128],
                 acc[NPAD:NPAD + N_NODES, :128]], axis=1)
            out_sum = out_sum + seg / counts[d_idx]
        xs.append(jnp.maximum(out_sum, 0.0))

    cat = jnp.concatenate(xs, axis=-1)
    out = _matmul_nt(cat, params['final_W'], bm=400) + params['final_b']
    return out


# speedup vs baseline: 1.7231x; 1.0394x over previous
import functools
from functools import partial

import jax
import jax.numpy as jnp
from jax import lax
from jax.experimental import pallas as pl
from jax.experimental.pallas import tpu as pltpu
from jax.experimental.pallas import tpu_sc as plsc

N_NODES = 10000
N_EDGES = 160000
D = 256
D_EDGE = 768
BATCH_SIZE = 128.0
EPS = 1e-08

NC = 2
NS = 16
NW = NC * NS

CH = 128
HF = 64
NCHUNK = N_EDGES // CH
NPAD = 10240
ACC_W = 128


def _sc_mesh():
    return plsc.VectorSubcoreMesh(
        core_axis_name="c", subcore_axis_name="s",
        num_cores=NC, num_subcores=NS)


def _mm_nt_body(a_ref, b_ref, o_ref):
    o_ref[...] = jax.lax.dot_general(
        a_ref[...], b_ref[...],
        dimension_numbers=(((1,), (1,)), ((), ())),
        preferred_element_type=jnp.float32)


def _matmul_nt(a, b, bm):
    m, k = a.shape
    n = b.shape[0]
    grid = (m // bm,)
    return pl.pallas_call(
        _mm_nt_body,
        grid=grid,
        in_specs=[pl.BlockSpec((bm, k), lambda i: (i, 0)),
                  pl.BlockSpec((n, k), lambda i: (0, 0))],
        out_specs=pl.BlockSpec((bm, n), lambda i: (i, 0)),
        out_shape=jax.ShapeDtypeStruct((m, n), jnp.float32),
    )(a, b)


P1_ITER = (NCHUNK + NW - 1) // NW


def _phase1_body(pi_h, pj_h, e_h, ii_h, ij_h, w1_h,
                 t_h,
                 ii_c, ij_c, gi0, gj0, ev0, gi1, gj1, ev1,
                 w1_v, tr_v, t_c, sem0, sem1, sem_t):
    cid = lax.axis_index("c")
    sid = lax.axis_index("s")
    wid = cid * NS + sid

    pltpu.sync_copy(w1_h, w1_v)
    lane = lax.broadcasted_iota(jnp.int32, (16,), 0)

    def issue(cb, half, gi_v, gj_v, ev_v, sem):
        pltpu.async_copy(pi_h.at[ii_c.at[pl.ds(half * HF, HF)]], gi_v, sem)
        pltpu.async_copy(pj_h.at[ij_c.at[pl.ds(half * HF, HF)]], gj_v, sem)
        pltpu.async_copy(e_h.at[pl.ds(cb + half * HF, HF)], ev_v, sem)

    def wait(gi_v, gj_v, ev_v, sem):
        for dst in (gi_v, gj_v, ev_v):
            pltpu.make_async_copy(e_h.at[pl.ds(0, HF)], dst, sem).wait()

    def compute(half, gi_v, gj_v, ev_v):
        @pl.loop(0, HF // 16)
        def _grp(grp):
            @pl.loop(0, 16)
            def _edge(j):
                e = grp * 16 + j
                acc = jnp.zeros((16,), jnp.float32)
                for k in range(D // 16):
                    v = (gi_v[e, pl.ds(k * 16, 16)]
                         + gj_v[e, pl.ds(k * 16, 16)]
                         + ev_v[e, pl.ds(k * 16, 16)])
                    acc = acc + jnp.maximum(v, 0.0) * w1_v[pl.ds(k * 16, 16)]
                plsc.store_scatter(tr_v, [lane * 16 + j], acc)
            s = tr_v[pl.ds(0, 16)]
            for k in range(1, 16):
                s = s + tr_v[pl.ds(k * 16, 16)]
            plsc.store_scatter(
                t_c, [jnp.full((16,), half * HF + grp * 16, jnp.int32)
                      + lane], jnp.exp(s))

    @pl.loop(0, P1_ITER)
    def _chunk(i):
        ch = wid + i * NW

        @pl.when(ch < NCHUNK)
        def _():
            cb = ch * CH
            pltpu.sync_copy(ii_h.at[pl.ds(cb, CH)], ii_c)
            pltpu.sync_copy(ij_h.at[pl.ds(cb, CH)], ij_c)
            issue(cb, 0, gi0, gj0, ev0, sem0)
            issue(cb, 1, gi1, gj1, ev1, sem1)

            @pl.when(i > 0)
            def _():
                pltpu.make_async_copy(ii_h.at[pl.ds(0, CH)], t_c,
                                      sem_t).wait()

            wait(gi0, gj0, ev0, sem0)
            compute(0, gi0, gj0, ev0)
            wait(gi1, gj1, ev1, sem1)
            compute(1, gi1, gj1, ev1)
            pltpu.async_copy(t_c, t_h.at[pl.ds(cb, CH)], sem_t)

    pltpu.make_async_copy(ii_h.at[pl.ds(0, CH)], t_c, sem_t).wait()


def _phase1_call(pi, pj, e, idx_i, idx_j, w1vec):
    kern = pl.kernel(
        _phase1_body,
        out_type=jax.ShapeDtypeStruct((N_EDGES,), jnp.float32),
        mesh=_sc_mesh(),
        compiler_params=pltpu.CompilerParams(needs_layout_passes=False),
        scratch_types=[
            pltpu.VMEM((CH,), jnp.int32),
            pltpu.VMEM((CH,), jnp.int32),
            pltpu.VMEM((HF, D), jnp.float32),
            pltpu.VMEM((HF, D), jnp.float32),
            pltpu.VMEM((HF, D), jnp.float32),
            pltpu.VMEM((HF, D), jnp.float32),
            pltpu.VMEM((HF, D), jnp.float32),
            pltpu.VMEM((HF, D), jnp.float32),
            pltpu.VMEM((D,), jnp.float32),
            pltpu.VMEM((256,), jnp.float32),
            pltpu.VMEM((CH,), jnp.float32),
            pltpu.SemaphoreType.DMA,
            pltpu.SemaphoreType.DMA,
            pltpu.SemaphoreType.DMA,
        ],
    )
    return kern(pi, pj, e, idx_i, idx_j, w1vec)


P2_ITER = (NCHUNK + NS - 1) // NS


def _phase2_body(hcat_h, coef_h, ii_h, ij_h,
                 acc_h,
                 ii0, ii1, ij_c, cf_c, hv0, hv1, rb0, rb1,
                 acc_sp, semh0, semh1, sems0, sems1):
    cid = lax.axis_index("c")
    sid = lax.axis_index("s")

    rows = NPAD // NS
    zvec = jnp.zeros((16,), jnp.float32)

    @pl.loop(0, HF)
    def _zrow(e):
        for k in range(ACC_W // 16):
            rb0[e, pl.ds(k * 16, 16)] = zvec

    for r in range(rows // HF):
        pltpu.sync_copy(rb0, acc_sp.at[pl.ds(sid * rows + r * HF, HF)])

    plsc.subcore_barrier()

    off = jnp.full((16,), cid * N_NODES, jnp.int32)

    def compute(hv, rb, cf_base):
        @pl.loop(0, HF)
        def _edge(e):
            cvec = plsc.load_gather(
                cf_c, [jnp.full((16,), cf_base + e, jnp.int32)])
            for k in range(ACC_W // 16):
                rb[e, pl.ds(k * 16, 16)] = hv[e, pl.ds(k * 16, 16)] * cvec

    def drain_sc(rb, sems):
        pltpu.make_async_copy(hcat_h.at[pl.ds(0, HF)], rb, sems).wait()

    @pl.loop(0, P2_ITER)
    def _chunk(i):
        ch = sid + i * NS

        @pl.when(ch < NCHUNK)
        def _():
            cb = ch * CH
            pltpu.sync_copy(ij_h.at[pl.ds(cb, CH)], ij_c)
            pltpu.sync_copy(coef_h.at[pl.ds(cb, CH)], cf_c)
            pltpu.async_copy(ii_h.at[pl.ds(cb, HF)], ii0, semh0)
            pltpu.async_copy(ii_h.at[pl.ds(cb + HF, HF)], ii1, semh1)
            for r in range(CH // 16):
                ij_c[pl.ds(r * 16, 16)] = ij_c[pl.ds(r * 16, 16)] + off
            pltpu.async_copy(hcat_h.at[ij_c.at[pl.ds(0, HF)]], hv0, semh0)
            pltpu.async_copy(hcat_h.at[ij_c.at[pl.ds(HF, HF)]], hv1, semh1)

            pltpu.make_async_copy(ii_h.at[pl.ds(0, HF)], ii0, semh0).wait()
            pltpu.make_async_copy(hcat_h.at[pl.ds(0, HF)], hv0,
                                  semh0).wait()

            @pl.when(i > 0)
            def _():
                drain_sc(rb0, sems0)

            compute(hv0, rb0, 0)
            pltpu.async_copy(rb0, acc_sp.at[ii0], sems0, add=True)

            pltpu.make_async_copy(ii_h.at[pl.ds(0, HF)], ii1, semh1).wait()
            pltpu.make_async_copy(hcat_h.at[pl.ds(0, HF)], hv1,
                                  semh1).wait()

            @pl.when(i > 0)
            def _():
                drain_sc(rb1, sems1)

            compute(hv1, rb1, HF)
            pltpu.async_copy(rb1, acc_sp.at[ii1], sems1, add=True)

    drain_sc(rb0, sems0)
    drain_sc(rb1, sems1)
    plsc.subcore_barrier()

    pltpu.sync_copy(acc_sp.at[pl.ds(sid * rows, rows)],
                    acc_h.at[pl.ds(cid * NPAD + sid * rows, rows)])


def _phase2_call(hcat, coef, idx_i, idx_j):
    kern = pl.kernel(
        _phase2_body,
        out_type=jax.ShapeDtypeStruct((NC * NPAD, ACC_W), jnp.float32),
        mesh=_sc_mesh(),
        compiler_params=pltpu.CompilerParams(needs_layout_passes=False),
        scratch_types=[
            pltpu.VMEM((HF,), jnp.int32),
            pltpu.VMEM((HF,), jnp.int32),
            pltpu.VMEM((CH,), jnp.int32),
            pltpu.VMEM((CH,), jnp.float32),
            pltpu.VMEM((HF, 128), jnp.float32),
            pltpu.VMEM((HF, 128), jnp.float32),
            pltpu.VMEM((HF, ACC_W), jnp.float32),
            pltpu.VMEM((HF, ACC_W), jnp.float32),
            pltpu.VMEM_SHARED((NPAD, ACC_W), jnp.float32),
            pltpu.SemaphoreType.DMA,
            pltpu.SemaphoreType.DMA,
            pltpu.SemaphoreType.DMA,
            pltpu.SemaphoreType.DMA,
        ],
    )
    return kern(hcat, coef, idx_i, idx_j)


def kernel(x, edge_index, edge_attr, params):
    src = edge_index[0]
    dst = edge_index[1]
    n = x.shape[0]

    layer_w = []
    for p in params['layers']:
        dirs = []
        for (lin, line, w0, w1) in (
            (p['lin1'], p['lin3'], p['lin5_0'], p['lin5_1']),
            (p['lin2'], p['lin4'], p['lin6_0'], p['lin6_1']),
        ):
            ci = w0[:, :D] @ lin
            cj = w0[:, D:2 * D] @ lin
            ce = w0[:, 2 * D:] @ line
            dirs.append({'lin': lin, 'ci': ci, 'cj': cj, 'ce': ce,
                         'w1': w1[0]})
        layer_w.append(dirs)

    e_l = [
        _matmul_nt(edge_attr,
                   jnp.concatenate([layer_w[l][0]['ce'],
                                    layer_w[l][1]['ce']], axis=0),
                   bm=640)
        for l in range(len(params['layers']))
    ]

    ones = jnp.ones((N_EDGES, 1), jnp.float32)
    counts = [jnp.maximum(jax.ops.segment_sum(ones, dst, num_segments=n), 1.0),
              jnp.maximum(jax.ops.segment_sum(ones, src, num_segments=n), 1.0)]

    xs = [x]
    for l, p in enumerate(params['layers']):
        xin = xs[-1]
        dirs = layer_w[l]
        wn = jnp.concatenate([
            dirs[0]['lin'], dirs[0]['ci'], dirs[0]['cj'],
            dirs[1]['lin'], dirs[1]['ci'], dirs[1]['cj'],
            p['root_W'],
        ], axis=0)
        proj = _matmul_nt(xin, wn, bm=400)
        h1 = proj[:, 0:D]
        p1i = proj[:, D:2 * D]
        p1j = proj[:, 2 * D:3 * D]
        h2 = proj[:, 3 * D:4 * D]
        p2i = proj[:, 4 * D:5 * D]
        p2j = proj[:, 5 * D:6 * D]
        root = proj[:, 6 * D:7 * D]

        out_sum = root + p['root_b']
        for d_idx, (h, pi, pj, idx_i, idx_j) in enumerate((
                (h1, p1i, p1j, dst, src),
                (h2, p2i, p2j, src, dst))):
            e = jax.lax.dynamic_slice_in_dim(e_l[l], d_idx * D, D, 1)
            t = _phase1_call(pi, pj, e, idx_i, idx_j, dirs[d_idx]['w1'])
            add = jnp.sum(t) / BATCH_SIZE
            add = jnp.where(add == 0.0, EPS, add)
            coef = jnp.minimum(t / add, 1.0) * p['W']
            hcat = jnp.concatenate([h[:, :128], h[:, 128:]], axis=0)
            acc = _phase2_call(hcat, coef, idx_i, idx_j)
            seg = jnp.concatenate(
                [acc[:N_NODES, :128],
                 acc[NPAD:NPAD + N_NODES, :128]], axis=1)
            out_sum = out_sum + seg / counts[d_idx]
        xs.append(jnp.maximum(out_sum, 0.0))

    cat = jnp.concatenate(xs, axis=-1)
    out = _matmul_nt(cat, params['final_W'], bm=400) + params['final_b']
    return out


# speedup vs baseline: 1.9391x; 1.1254x over previous
import functools
from functools import partial

import jax
import jax.numpy as jnp
from jax import lax
from jax.experimental import pallas as pl
from jax.experimental.pallas import tpu as pltpu
from jax.experimental.pallas import tpu_sc as plsc

N_NODES = 10000
N_EDGES = 160000
D = 256
D_EDGE = 768
BATCH_SIZE = 128.0
EPS = 1e-08

NC = 2
NS = 16
NW = NC * NS

CH = 128
HF = 64
NCHUNK = N_EDGES // CH
NPAD = 10240
ACC_W = 128


def _sc_mesh():
    return plsc.VectorSubcoreMesh(
        core_axis_name="c", subcore_axis_name="s",
        num_cores=NC, num_subcores=NS)


def _mm_nt_body(a_ref, b_ref, o_ref):
    o_ref[...] = jax.lax.dot_general(
        a_ref[...], b_ref[...],
        dimension_numbers=(((1,), (1,)), ((), ())),
        preferred_element_type=jnp.float32)


def _matmul_nt(a, b, bm):
    m, k = a.shape
    n = b.shape[0]
    grid = (m // bm,)
    return pl.pallas_call(
        _mm_nt_body,
        grid=grid,
        in_specs=[pl.BlockSpec((bm, k), lambda i: (i, 0)),
                  pl.BlockSpec((n, k), lambda i: (0, 0))],
        out_specs=pl.BlockSpec((bm, n), lambda i: (i, 0)),
        out_shape=jax.ShapeDtypeStruct((m, n), jnp.float32),
    )(a, b)


P1_UNITS = N_EDGES // HF
P1_UITER = (P1_UNITS + NW - 1) // NW


def _phase1_body(pi_h, pj_h, e_h, ii_h, ij_h, w1_h,
                 t_h,
                 iiA, ijA, iiB, ijB, giA, gjA, evA, giB, gjB, evB,
                 w1_v, tr_v, tA, tB, semA, semB, semTA, semTB):
    cid = lax.axis_index("c")
    sid = lax.axis_index("s")
    wid = cid * NS + sid

    pltpu.sync_copy(w1_h, w1_v)
    lane = lax.broadcasted_iota(jnp.int32, (16,), 0)

    def loadidx(u, ii_c, ij_c):
        pltpu.sync_copy(ii_h.at[pl.ds(u * HF, HF)], ii_c)
        pltpu.sync_copy(ij_h.at[pl.ds(u * HF, HF)], ij_c)

    def issue(u, ii_c, ij_c, gi, gj, ev, sem):
        pltpu.async_copy(pi_h.at[ii_c], gi, sem)
        pltpu.async_copy(pj_h.at[ij_c], gj, sem)
        pltpu.async_copy(e_h.at[pl.ds(u * HF, HF)], ev, sem)

    def wait3(gi, gj, ev, sem):
        for dst in (gi, gj, ev):
            pltpu.make_async_copy(e_h.at[pl.ds(0, HF)], dst, sem).wait()

    def compute(gi_v, gj_v, ev_v, t_c):
        @pl.loop(0, HF // 16)
        def _grp(grp):
            @pl.loop(0, 16)
            def _edge(j):
                e = grp * 16 + j
                acc = jnp.zeros((16,), jnp.float32)
                for k in range(D // 16):
                    v = (gi_v[e, pl.ds(k * 16, 16)]
                         + gj_v[e, pl.ds(k * 16, 16)]
                         + ev_v[e, pl.ds(k * 16, 16)])
                    acc = acc + jnp.maximum(v, 0.0) * w1_v[pl.ds(k * 16, 16)]
                plsc.store_scatter(tr_v, [lane * 16 + j], acc)
            s = tr_v[pl.ds(0, 16)]
            for k in range(1, 16):
                s = s + tr_v[pl.ds(k * 16, 16)]
            plsc.store_scatter(
                t_c, [jnp.full((16,), grp * 16, jnp.int32) + lane],
                jnp.exp(s))

    def tdrain(t_c, semT):
        pltpu.make_async_copy(ii_h.at[pl.ds(0, HF)], t_c, semT).wait()

    uA0 = wid
    uB0 = wid + NW
    loadidx(uA0, iiA, ijA)
    issue(uA0, iiA, ijA, giA, gjA, evA, semA)
    loadidx(uB0, iiB, ijB)
    issue(uB0, iiB, ijB, giB, gjB, evB, semB)

    @pl.loop(0, (P1_UITER + 1) // 2)
    def _pair(pp):
        uA = wid + (2 * pp) * NW
        uB = wid + (2 * pp + 1) * NW
        uA2 = wid + (2 * pp + 2) * NW
        uB2 = wid + (2 * pp + 3) * NW

        @pl.when(uA < P1_UNITS)
        def _():
            wait3(giA, gjA, evA, semA)

            @pl.when(pp > 0)
            def _():
                tdrain(tA, semTA)

            compute(giA, gjA, evA, tA)
            pltpu.async_copy(tA, t_h.at[pl.ds(uA * HF, HF)], semTA)

            @pl.when(uA2 < P1_UNITS)
            def _():
                loadidx(uA2, iiA, ijA)
                issue(uA2, iiA, ijA, giA, gjA, evA, semA)

        @pl.when(uB < P1_UNITS)
        def _():
            wait3(giB, gjB, evB, semB)

            @pl.when(pp > 0)
            def _():
                tdrain(tB, semTB)

            compute(giB, gjB, evB, tB)
            pltpu.async_copy(tB, t_h.at[pl.ds(uB * HF, HF)], semTB)

            @pl.when(uB2 < P1_UNITS)
            def _():
                loadidx(uB2, iiB, ijB)
                issue(uB2, iiB, ijB, giB, gjB, evB, semB)

    tdrain(tA, semTA)
    tdrain(tB, semTB)


def _phase1_call(pi, pj, e, idx_i, idx_j, w1vec):
    kern = pl.kernel(
        _phase1_body,
        out_type=jax.ShapeDtypeStruct((N_EDGES,), jnp.float32),
        mesh=_sc_mesh(),
        compiler_params=pltpu.CompilerParams(needs_layout_passes=False),
        scratch_types=[
            pltpu.VMEM((HF,), jnp.int32),
            pltpu.VMEM((HF,), jnp.int32),
            pltpu.VMEM((HF,), jnp.int32),
            pltpu.VMEM((HF,), jnp.int32),
            pltpu.VMEM((HF, D), jnp.float32),
            pltpu.VMEM((HF, D), jnp.float32),
            pltpu.VMEM((HF, D), jnp.float32),
            pltpu.VMEM((HF, D), jnp.float32),
            pltpu.VMEM((HF, D), jnp.float32),
            pltpu.VMEM((HF, D), jnp.float32),
            pltpu.VMEM((D,), jnp.float32),
            pltpu.VMEM((256,), jnp.float32),
            pltpu.VMEM((HF,), jnp.float32),
            pltpu.VMEM((HF,), jnp.float32),
            pltpu.SemaphoreType.DMA,
            pltpu.SemaphoreType.DMA,
            pltpu.SemaphoreType.DMA,
            pltpu.SemaphoreType.DMA,
        ],
    )
    return kern(pi, pj, e, idx_i, idx_j, w1vec)


P2_ITER = (NCHUNK + NS - 1) // NS


def _phase2_body(hcat_h, coef_h, ii_h, ij_h,
                 acc_h,
                 ii0, ii1, ij_c, cf_c, hv0, hv1, rb0, rb1,
                 acc_sp, semh0, semh1, sems0, sems1):
    cid = lax.axis_index("c")
    sid = lax.axis_index("s")

    rows = NPAD // NS
    zvec = jnp.zeros((16,), jnp.float32)

    @pl.loop(0, HF)
    def _zrow(e):
        for k in range(ACC_W // 16):
            rb0[e, pl.ds(k * 16, 16)] = zvec

    for r in range(rows // HF):
        pltpu.sync_copy(rb0, acc_sp.at[pl.ds(sid * rows + r * HF, HF)])

    plsc.subcore_barrier()

    off = jnp.full((16,), cid * N_NODES, jnp.int32)

    def compute(hv, rb, cf_base):
        @pl.loop(0, HF)
        def _edge(e):
            cvec = plsc.load_gather(
                cf_c, [jnp.full((16,), cf_base + e, jnp.int32)])
            for k in range(ACC_W // 16):
                rb[e, pl.ds(k * 16, 16)] = hv[e, pl.ds(k * 16, 16)] * cvec

    def drain_sc(rb, sems):
        pltpu.make_async_copy(hcat_h.at[pl.ds(0, HF)], rb, sems).wait()

    @pl.loop(0, P2_ITER)
    def _chunk(i):
        ch = sid + i * NS

        @pl.when(ch < NCHUNK)
        def _():
            cb = ch * CH
            pltpu.sync_copy(ij_h.at[pl.ds(cb, CH)], ij_c)
            pltpu.sync_copy(coef_h.at[pl.ds(cb, CH)], cf_c)
            pltpu.async_copy(ii_h.at[pl.ds(cb, HF)], ii0, semh0)
            pltpu.async_copy(ii_h.at[pl.ds(cb + HF, HF)], ii1, semh1)
            for r in range(CH // 16):
                ij_c[pl.ds(r * 16, 16)] = ij_c[pl.ds(r * 16, 16)] + off
            pltpu.async_copy(hcat_h.at[ij_c.at[pl.ds(0, HF)]], hv0, semh0)
            pltpu.async_copy(hcat_h.at[ij_c.at[pl.ds(HF, HF)]], hv1, semh1)

            pltpu.make_async_copy(ii_h.at[pl.ds(0, HF)], ii0, semh0).wait()
            pltpu.make_async_copy(hcat_h.at[pl.ds(0, HF)], hv0,
                                  semh0).wait()

            @pl.when(i > 0)
            def _():
                drain_sc(rb0, sems0)

            compute(hv0, rb0, 0)
            pltpu.async_copy(rb0, acc_sp.at[ii0], sems0, add=True)

            pltpu.make_async_copy(ii_h.at[pl.ds(0, HF)], ii1, semh1).wait()
            pltpu.make_async_copy(hcat_h.at[pl.ds(0, HF)], hv1,
                                  semh1).wait()

            @pl.when(i > 0)
            def _():
                drain_sc(rb1, sems1)

            compute(hv1, rb1, HF)
            pltpu.async_copy(rb1, acc_sp.at[ii1], sems1, add=True)

    drain_sc(rb0, sems0)
    drain_sc(rb1, sems1)
    plsc.subcore_barrier()

    pltpu.sync_copy(acc_sp.at[pl.ds(sid * rows, rows)],
                    acc_h.at[pl.ds(cid * NPAD + sid * rows, rows)])


def _phase2_call(hcat, coef, idx_i, idx_j):
    kern = pl.kernel(
        _phase2_body,
        out_type=jax.ShapeDtypeStruct((NC * NPAD, ACC_W), jnp.float32),
        mesh=_sc_mesh(),
        compiler_params=pltpu.CompilerParams(needs_layout_passes=False),
        scratch_types=[
            pltpu.VMEM((HF,), jnp.int32),
            pltpu.VMEM((HF,), jnp.int32),
            pltpu.VMEM((CH,), jnp.int32),
            pltpu.VMEM((CH,), jnp.float32),
            pltpu.VMEM((HF, 128), jnp.float32),
            pltpu.VMEM((HF, 128), jnp.float32),
            pltpu.VMEM((HF, ACC_W), jnp.float32),
            pltpu.VMEM((HF, ACC_W), jnp.float32),
            pltpu.VMEM_SHARED((NPAD, ACC_W), jnp.float32),
            pltpu.SemaphoreType.DMA,
            pltpu.SemaphoreType.DMA,
            pltpu.SemaphoreType.DMA,
            pltpu.SemaphoreType.DMA,
        ],
    )
    return kern(hcat, coef, idx_i, idx_j)


def kernel(x, edge_index, edge_attr, params):
    src = edge_index[0]
    dst = edge_index[1]
    n = x.shape[0]

    layer_w = []
    for p in params['layers']:
        dirs = []
        for (lin, line, w0, w1) in (
            (p['lin1'], p['lin3'], p['lin5_0'], p['lin5_1']),
            (p['lin2'], p['lin4'], p['lin6_0'], p['lin6_1']),
        ):
            ci = w0[:, :D] @ lin
            cj = w0[:, D:2 * D] @ lin
            ce = w0[:, 2 * D:] @ line
            dirs.append({'lin': lin, 'ci': ci, 'cj': cj, 'ce': ce,
                         'w1': w1[0]})
        layer_w.append(dirs)

    e_l = [
        _matmul_nt(edge_attr,
                   jnp.concatenate([layer_w[l][0]['ce'],
                                    layer_w[l][1]['ce']], axis=0),
                   bm=640)
        for l in range(len(params['layers']))
    ]

    ones = jnp.ones((N_EDGES, 1), jnp.float32)
    counts = [jnp.maximum(jax.ops.segment_sum(ones, dst, num_segments=n), 1.0),
              jnp.maximum(jax.ops.segment_sum(ones, src, num_segments=n), 1.0)]

    xs = [x]
    for l, p in enumerate(params['layers']):
        xin = xs[-1]
        dirs = layer_w[l]
        wn = jnp.concatenate([
            dirs[0]['lin'], dirs[0]['ci'], dirs[0]['cj'],
            dirs[1]['lin'], dirs[1]['ci'], dirs[1]['cj'],
            p['root_W'],
        ], axis=0)
        proj = _matmul_nt(xin, wn, bm=400)
        h1 = proj[:, 0:D]
        p1i = proj[:, D:2 * D]
        p1j = proj[:, 2 * D:3 * D]
        h2 = proj[:, 3 * D:4 * D]
        p2i = proj[:, 4 * D:5 * D]
        p2j = proj[:, 5 * D:6 * D]
        root = proj[:, 6 * D:7 * D]

        out_sum = root + p['root_b']
        for d_idx, (h, pi, pj, idx_i, idx_j) in enumerate((
                (h1, p1i, p1j, dst, src),
                (h2, p2i, p2j, src, dst))):
            e = jax.lax.dynamic_slice_in_dim(e_l[l], d_idx * D, D, 1)
            t = _phase1_call(pi, pj, e, idx_i, idx_j, dirs[d_idx]['w1'])
            add = jnp.sum(t) / BATCH_SIZE
            add = jnp.where(add == 0.0, EPS, add)
            coef = jnp.minimum(t / add, 1.0) * p['W']
            hcat = jnp.concatenate([h[:, :128], h[:, 128:]], axis=0)
            acc = _phase2_call(hcat, coef, idx_i, idx_j)
            seg = jnp.concatenate(
                [acc[:N_NODES, :128],
                 acc[NPAD:NPAD + N_NODES, :128]], axis=1)
            out_sum = out_sum + seg / counts[d_idx]
        xs.append(jnp.maximum(out_sum, 0.0))

    cat = jnp.concatenate(xs, axis=-1)
    out = _matmul_nt(cat, params['final_W'], bm=400) + params['final_b']
    return out
